# baseline (speedup 1.0000x reference)
import functools

import jax
import jax.numpy as jnp
from jax import lax
from jax.experimental import pallas as pl
from jax.experimental.pallas import tpu as pltpu
from jax.experimental.pallas import tpu_sc as plsc

B, W, H, C = 8, 224, 224, 96
LANES = 16
HCHUNK = H // LANES
NWORK = 32
NTASK = B * C
SC_NTASK = 512
TPW = SC_NTASK // NWORK
TC_JB = 16
TC_STEPS = (NTASK - SC_NTASK) // TC_JB
J_BLOCKS = C // TC_JB

_mesh = plsc.VectorSubcoreMesh(core_axis_name="c", subcore_axis_name="s")


@functools.partial(
    pl.kernel,
    mesh=_mesh,
    out_type=jax.ShapeDtypeStruct((SC_NTASK * 2,), jnp.float32),
    scratch_types=[
        pltpu.VMEM((C, H), jnp.float32),
        pltpu.VMEM((C, H), jnp.float32),
        pltpu.VMEM((C, H), jnp.float32),
        pltpu.VMEM((C, H), jnp.float32),
        pltpu.VMEM((2 * TPW,), jnp.float32),
        pltpu.SemaphoreType.DMA,
        pltpu.SemaphoreType.DMA,
        pltpu.SemaphoreType.DMA,
        pltpu.SemaphoreType.DMA,
    ],
)
def _argmax_sc(xt_hbm, out_hbm, buf0, buf1, buf2, buf3, obuf,
               sem0, sem1, sem2, sem3):
    cid = lax.axis_index("c")
    sid = lax.axis_index("s")
    wid = sid * 2 + cid
    t0 = wid * TPW

    bufs = (buf0, buf1, buf2, buf3)
    sems = (sem0, sem1, sem2, sem3)
    nbuf = len(bufs)

    def start_copy(k):
        t = t0 + k
        return pltpu.async_copy(xt_hbm.at[t // C, t % C], bufs[k % nbuf], sems[k % nbuf])

    copies = [start_copy(0), start_copy(1), start_copy(2), None]
    lanes = lax.iota(jnp.int32, LANES)
    acc = jnp.zeros((LANES,), jnp.float32)

    for k in range(TPW):
        if k + 3 < TPW:
            copies[(k + 3) % nbuf] = start_copy(k + 3)
        copies[k % nbuf].wait()
        buf = bufs[k % nbuf]

        def step(c, carry):
            new = []
            for g in range(HCHUNK):
                m, rc = carry[g]
                v = buf[c, pl.ds(g * LANES, LANES)]
                gt = v > m
                m = jnp.where(gt, v, m)
                rc = jnp.where(gt, c, rc)
                new.append((m, rc))
            return tuple(new)

        m0 = jnp.full((LANES,), -jnp.inf, jnp.float32)
        rc0 = jnp.zeros((LANES,), jnp.int32)
        accs = lax.fori_loop(0, C, step, tuple((m0, rc0) for _ in range(HCHUNK)))

        m, rc = accs[0]
        flat = rc + lanes * C
        for g in range(1, HCHUNK):
            mg, rcg = accs[g]
            fg = rcg + (lanes * C + g * (LANES * C))
            take = jnp.logical_or(mg > m, jnp.logical_and(mg == m, fg < flat))
            m = jnp.where(take, mg, m)
            flat = jnp.where(take, fg, flat)

        bv, bi = m[0], flat[0]
        for l in range(1, LANES):
            v, i = m[l], flat[l]
            take = jnp.logical_or(v > bv, jnp.logical_and(v == bv, i < bi))
            bv = jnp.where(take, v, bv)
            bi = jnp.where(take, i, bi)
        gidx = bi
        colf = (gidx % W).astype(jnp.float32)
        rowf = (gidx // W).astype(jnp.float32)
        p = (2 * k) % LANES
        acc = jnp.where(lanes == p, colf, acc)
        acc = jnp.where(lanes == p + 1, rowf, acc)
        if p + 2 == LANES:
            obuf[pl.ds((k // (LANES // 2)) * LANES, LANES)] = acc

    pltpu.sync_copy(obuf, out_hbm.at[pl.ds(t0 * 2, 2 * TPW)])


def _argmax_tc_body(xt_ref, o_ref):
    iota_c = lax.broadcasted_iota(jnp.int32, (C, H), 0).astype(jnp.float32)
    big = jnp.float32(2.0**30)
    colmaxs, argcs = [], []
    for b in range(TC_JB):
        arr = xt_ref[0, b]
        cm = jnp.max(arr, axis=0, keepdims=True)
        ac = jnp.min(jnp.where(arr == cm, iota_c, big), axis=0, keepdims=True)
        colmaxs.append(cm)
        argcs.append(ac)
    cmst = jnp.concatenate(colmaxs, axis=0)
    acst = jnp.concatenate(argcs, axis=0)
    ioh = lax.broadcasted_iota(jnp.int32, (TC_JB, H), 1).astype(jnp.float32)
    gmax = jnp.max(cmst, axis=1, keepdims=True)
    hmin = jnp.min(jnp.where(cmst == gmax, ioh, big), axis=1, keepdims=True)
    cmin = jnp.min(jnp.where(ioh == hmin, acst, big), axis=1, keepdims=True)
    flat = (hmin * C + cmin).astype(jnp.int32)
    colv = (flat % W).astype(jnp.float32)
    rowv = (flat // W).astype(jnp.float32)
    o_ref[0] = jnp.concatenate([colv, rowv], axis=1)


_argmax_tc = pl.pallas_call(
    _argmax_tc_body,
    grid=(TC_STEPS,),
    in_specs=[
        pl.BlockSpec(
            (1, TC_JB, C, H),
            lambda g: ((SC_NTASK // C + (SC_NTASK % C // TC_JB + g) // J_BLOCKS),
                       (SC_NTASK % C // TC_JB + g) % J_BLOCKS, 0, 0),
        )
    ],
    out_specs=pl.BlockSpec((1, TC_JB, 2), lambda g: (g, 0, 0)),
    out_shape=jax.ShapeDtypeStruct((TC_STEPS, TC_JB, 2), jnp.float32),
)


def kernel(x):
    xt = jnp.transpose(x, (0, 1, 3, 2))
    sc_out = _argmax_sc(xt)
    tc_out = _argmax_tc(xt)
    return jnp.concatenate([sc_out, tc_out.reshape(-1)]).reshape(B, 2 * C)


# speedup vs baseline: 1.0224x; 1.0224x over previous
import functools

import jax
import jax.numpy as jnp
from jax import lax
from jax.experimental import pallas as pl
from jax.experimental.pallas import tpu as pltpu
from jax.experimental.pallas import tpu_sc as plsc

B, W, H, C = 8, 224, 224, 96
LANES = 16
HCHUNK = H // LANES
NWORK = 32
NTASK = B * C
SC_NTASK = 512
TPW = SC_NTASK // NWORK
TC_JB = 32
TC_STEPS = (NTASK - SC_NTASK) // TC_JB
J_BLOCKS = C // TC_JB

_mesh = plsc.VectorSubcoreMesh(core_axis_name="c", subcore_axis_name="s")


@functools.partial(
    pl.kernel,
    mesh=_mesh,
    out_type=jax.ShapeDtypeStruct((SC_NTASK * 2,), jnp.float32),
    scratch_types=[
        pltpu.VMEM((C, H), jnp.float32),
        pltpu.VMEM((C, H), jnp.float32),
        pltpu.VMEM((C, H), jnp.float32),
        pltpu.VMEM((C, H), jnp.float32),
        pltpu.VMEM((2 * TPW,), jnp.float32),
        pltpu.SemaphoreType.DMA,
        pltpu.SemaphoreType.DMA,
        pltpu.SemaphoreType.DMA,
        pltpu.SemaphoreType.DMA,
    ],
)
def _argmax_sc(xt_hbm, out_hbm, buf0, buf1, buf2, buf3, obuf,
               sem0, sem1, sem2, sem3):
    cid = lax.axis_index("c")
    sid = lax.axis_index("s")
    wid = sid * 2 + cid
    t0 = wid * TPW

    bufs = (buf0, buf1, buf2, buf3)
    sems = (sem0, sem1, sem2, sem3)
    nbuf = len(bufs)

    def start_copy(k):
        t = t0 + k
        return pltpu.async_copy(xt_hbm.at[t // C, t % C], bufs[k % nbuf], sems[k % nbuf])

    copies = [start_copy(0), start_copy(1), start_copy(2), None]
    lanes = lax.iota(jnp.int32, LANES)
    acc = jnp.zeros((LANES,), jnp.float32)

    for k in range(TPW):
        if k + 3 < TPW:
            copies[(k + 3) % nbuf] = start_copy(k + 3)
        copies[k % nbuf].wait()
        buf = bufs[k % nbuf]

        def step(c, carry):
            new = []
            for g in range(HCHUNK):
                m, rc = carry[g]
                v = buf[c, pl.ds(g * LANES, LANES)]
                gt = v > m
                m = jnp.where(gt, v, m)
                rc = jnp.where(gt, c, rc)
                new.append((m, rc))
            return tuple(new)

        m0 = jnp.full((LANES,), -jnp.inf, jnp.float32)
        rc0 = jnp.zeros((LANES,), jnp.int32)
        accs = lax.fori_loop(0, C, step, tuple((m0, rc0) for _ in range(HCHUNK)))

        m, rc = accs[0]
        flat = rc + lanes * C
        for g in range(1, HCHUNK):
            mg, rcg = accs[g]
            fg = rcg + (lanes * C + g * (LANES * C))
            take = jnp.logical_or(mg > m, jnp.logical_and(mg == m, fg < flat))
            m = jnp.where(take, mg, m)
            flat = jnp.where(take, fg, flat)

        bv, bi = m[0], flat[0]
        for l in range(1, LANES):
            v, i = m[l], flat[l]
            take = jnp.logical_or(v > bv, jnp.logical_and(v == bv, i < bi))
            bv = jnp.where(take, v, bv)
            bi = jnp.where(take, i, bi)
        gidx = bi
        colf = (gidx % W).astype(jnp.float32)
        rowf = (gidx // W).astype(jnp.float32)
        p = (2 * k) % LANES
        acc = jnp.where(lanes == p, colf, acc)
        acc = jnp.where(lanes == p + 1, rowf, acc)
        if p + 2 == LANES:
            obuf[pl.ds((k // (LANES // 2)) * LANES, LANES)] = acc

    pltpu.sync_copy(obuf, out_hbm.at[pl.ds(t0 * 2, 2 * TPW)])


def _argmax_tc_body(xt_ref, o_ref):
    iota_c = lax.broadcasted_iota(jnp.int32, (C, H), 0).astype(jnp.float32)
    big = jnp.float32(2.0**30)
    colmaxs, argcs = [], []
    for b in range(TC_JB):
        arr = xt_ref[0, b]
        cm = jnp.max(arr, axis=0, keepdims=True)
        ac = jnp.min(jnp.where(arr == cm, iota_c, big), axis=0, keepdims=True)
        colmaxs.append(cm)
        argcs.append(ac)
    cmst = jnp.concatenate(colmaxs, axis=0)
    acst = jnp.concatenate(argcs, axis=0)
    ioh = lax.broadcasted_iota(jnp.int32, (TC_JB, H), 1).astype(jnp.float32)
    gmax = jnp.max(cmst, axis=1, keepdims=True)
    hmin = jnp.min(jnp.where(cmst == gmax, ioh, big), axis=1, keepdims=True)
    cmin = jnp.min(jnp.where(ioh == hmin, acst, big), axis=1, keepdims=True)
    flat = (hmin * C + cmin).astype(jnp.int32)
    colv = (flat % W).astype(jnp.float32)
    rowv = (flat // W).astype(jnp.float32)
    o_ref[0] = jnp.concatenate([colv, rowv], axis=1)


_argmax_tc = pl.pallas_call(
    _argmax_tc_body,
    grid=(TC_STEPS,),
    in_specs=[
        pl.BlockSpec(
            (1, TC_JB, C, H),
            lambda g: ((SC_NTASK // C + (SC_NTASK % C // TC_JB + g) // J_BLOCKS),
                       (SC_NTASK % C // TC_JB + g) % J_BLOCKS, 0, 0),
        )
    ],
    out_specs=pl.BlockSpec((1, TC_JB, 2), lambda g: (g, 0, 0)),
    out_shape=jax.ShapeDtypeStruct((TC_STEPS, TC_JB, 2), jnp.float32),
)


def kernel(x):
    xt = jnp.transpose(x, (0, 1, 3, 2))
    sc_out = _argmax_sc(xt)
    tc_out = _argmax_tc(xt)
    return jnp.concatenate([sc_out, tc_out.reshape(-1)]).reshape(B, 2 * C)


# speedup vs baseline: 1.0752x; 1.0516x over previous
import functools

import jax
import jax.numpy as jnp
from jax import lax
from jax.experimental import pallas as pl
from jax.experimental.pallas import tpu as pltpu
from jax.experimental.pallas import tpu_sc as plsc

B, W, H, C = 8, 224, 224, 96
LANES = 16
HCHUNK = H // LANES
NWORK = 32
NTASK = B * C
SC_NTASK = 384
TPW = SC_NTASK // NWORK
TC_JB = 32
TC_STEPS = (NTASK - SC_NTASK) // TC_JB
J_BLOCKS = C // TC_JB

_mesh = plsc.VectorSubcoreMesh(core_axis_name="c", subcore_axis_name="s")


@functools.partial(
    pl.kernel,
    mesh=_mesh,
    out_type=jax.ShapeDtypeStruct((SC_NTASK * 2,), jnp.float32),
    scratch_types=[
        pltpu.VMEM((C, H), jnp.float32),
        pltpu.VMEM((C, H), jnp.float32),
        pltpu.VMEM((C, H), jnp.float32),
        pltpu.VMEM((C, H), jnp.float32),
        pltpu.VMEM((((2 * TPW + LANES - 1) // LANES) * LANES,), jnp.float32),
        pltpu.SemaphoreType.DMA,
        pltpu.SemaphoreType.DMA,
        pltpu.SemaphoreType.DMA,
        pltpu.SemaphoreType.DMA,
    ],
)
def _argmax_sc(xt_hbm, out_hbm, buf0, buf1, buf2, buf3, obuf,
               sem0, sem1, sem2, sem3):
    cid = lax.axis_index("c")
    sid = lax.axis_index("s")
    wid = sid * 2 + cid
    t0 = wid * TPW

    bufs = (buf0, buf1, buf2, buf3)
    sems = (sem0, sem1, sem2, sem3)
    nbuf = len(bufs)

    def start_copy(k):
        t = t0 + k
        return pltpu.async_copy(xt_hbm.at[t // C, t % C], bufs[k % nbuf], sems[k % nbuf])

    copies = [start_copy(0), start_copy(1), start_copy(2), None]
    lanes = lax.iota(jnp.int32, LANES)
    acc = jnp.zeros((LANES,), jnp.float32)

    for k in range(TPW):
        if k + 3 < TPW:
            copies[(k + 3) % nbuf] = start_copy(k + 3)
        copies[k % nbuf].wait()
        buf = bufs[k % nbuf]

        def step(c, carry):
            new = []
            for g in range(HCHUNK):
                m, rc = carry[g]
                v = buf[c, pl.ds(g * LANES, LANES)]
                gt = v > m
                m = jnp.where(gt, v, m)
                rc = jnp.where(gt, c, rc)
                new.append((m, rc))
            return tuple(new)

        m0 = jnp.full((LANES,), -jnp.inf, jnp.float32)
        rc0 = jnp.zeros((LANES,), jnp.int32)
        accs = lax.fori_loop(0, C, step, tuple((m0, rc0) for _ in range(HCHUNK)))

        m, rc = accs[0]
        flat = rc + lanes * C
        for g in range(1, HCHUNK):
            mg, rcg = accs[g]
            fg = rcg + (lanes * C + g * (LANES * C))
            take = jnp.logical_or(mg > m, jnp.logical_and(mg == m, fg < flat))
            m = jnp.where(take, mg, m)
            flat = jnp.where(take, fg, flat)

        bv, bi = m[0], flat[0]
        for l in range(1, LANES):
            v, i = m[l], flat[l]
            take = jnp.logical_or(v > bv, jnp.logical_and(v == bv, i < bi))
            bv = jnp.where(take, v, bv)
            bi = jnp.where(take, i, bi)
        gidx = bi
        colf = (gidx % W).astype(jnp.float32)
        rowf = (gidx // W).astype(jnp.float32)
        p = (2 * k) % LANES
        acc = jnp.where(lanes == p, colf, acc)
        acc = jnp.where(lanes == p + 1, rowf, acc)
        if p + 2 == LANES:
            obuf[pl.ds((k // (LANES // 2)) * LANES, LANES)] = acc

    if (2 * TPW) % LANES != 0:
        obuf[pl.ds(((2 * TPW) // LANES) * LANES, LANES)] = acc
    pltpu.sync_copy(obuf.at[pl.ds(0, 2 * TPW)], out_hbm.at[pl.ds(t0 * 2, 2 * TPW)])


def _argmax_tc_body(xt_ref, o_ref):
    iota_c = lax.broadcasted_iota(jnp.int32, (C, H), 0).astype(jnp.float32)
    big = jnp.float32(2.0**30)
    colmaxs, argcs = [], []
    for b in range(TC_JB):
        arr = xt_ref[0, b]
        cm = jnp.max(arr, axis=0, keepdims=True)
        ac = jnp.min(jnp.where(arr == cm, iota_c, big), axis=0, keepdims=True)
        colmaxs.append(cm)
        argcs.append(ac)
    cmst = jnp.concatenate(colmaxs, axis=0)
    acst = jnp.concatenate(argcs, axis=0)
    ioh = lax.broadcasted_iota(jnp.int32, (TC_JB, H), 1).astype(jnp.float32)
    gmax = jnp.max(cmst, axis=1, keepdims=True)
    hmin = jnp.min(jnp.where(cmst == gmax, ioh, big), axis=1, keepdims=True)
    cmin = jnp.min(jnp.where(ioh == hmin, acst, big), axis=1, keepdims=True)
    flat = (hmin * C + cmin).astype(jnp.int32)
    colv = (flat % W).astype(jnp.float32)
    rowv = (flat // W).astype(jnp.float32)
    o_ref[0] = jnp.concatenate([colv, rowv], axis=1)


_argmax_tc = pl.pallas_call(
    _argmax_tc_body,
    grid=(TC_STEPS,),
    in_specs=[
        pl.BlockSpec(
            (1, TC_JB, C, H),
            lambda g: ((SC_NTASK // C + (SC_NTASK % C // TC_JB + g) // J_BLOCKS),
                       (SC_NTASK % C // TC_JB + g) % J_BLOCKS, 0, 0),
        )
    ],
    out_specs=pl.BlockSpec((1, TC_JB, 2), lambda g: (g, 0, 0)),
    out_shape=jax.ShapeDtypeStruct((TC_STEPS, TC_JB, 2), jnp.float32),
)


def kernel(x):
    xt = jnp.transpose(x, (0, 1, 3, 2))
    sc_out = _argmax_sc(xt)
    tc_out = _argmax_tc(xt)
    return jnp.concatenate([sc_out, tc_out.reshape(-1)]).reshape(B, 2 * C)
